# baseline (speedup 1.0000x reference)
import jax
import jax.numpy as jnp
from jax.experimental import pallas as pl
from jax.experimental.pallas import tpu as pltpu

N = 4096
D = 128
R = 2
BM = 256


def _body(feat_ref, adj_ref, adjt_ref, w_ref, b_ref, wt_ref, bt_ref,
          out_ref, g_scr, gt_scr, bw_scr):
    m = pl.program_id(0)

    @pl.when(m == 0)
    def _prologue():
        f = feat_ref[...].astype(jnp.bfloat16)
        for r in range(R):
            g_scr[r] = jnp.dot(f, w_ref[r].astype(jnp.bfloat16),
                               preferred_element_type=jnp.float32).astype(jnp.bfloat16)
            gt_scr[r] = jnp.dot(f, wt_ref[r].astype(jnp.bfloat16),
                                preferred_element_type=jnp.float32).astype(jnp.bfloat16)
            bw_scr[pl.ds(r, 1), :] = jnp.dot(
                b_ref[pl.ds(r, 1), :], w_ref[r],
                preferred_element_type=jnp.float32)
            bw_scr[pl.ds(R + r, 1), :] = jnp.dot(
                bt_ref[pl.ds(r, 1), :], wt_ref[r],
                preferred_element_type=jnp.float32)

    acc = jnp.zeros((BM, D), jnp.float32)
    cnt = jnp.zeros((BM, 1), jnp.float32)
    for r in range(R):
        a = adj_ref[r]
        at = adjt_ref[r]
        y = jnp.dot(a.astype(jnp.bfloat16), g_scr[r],
                    preferred_element_type=jnp.float32)
        yt = jnp.dot(at.astype(jnp.bfloat16), gt_scr[r],
                     preferred_element_type=jnp.float32)
        deg = jnp.sum(a, axis=1, keepdims=True).astype(jnp.float32)
        degt = jnp.sum(at, axis=1, keepdims=True).astype(jnp.float32)
        mask = (deg > 0.0).astype(jnp.float32)
        maskt = (degt > 0.0).astype(jnp.float32)
        acc = acc + (y / jnp.maximum(deg, 1.0)
                     + mask * bw_scr[pl.ds(r, 1), :]
                     + yt / jnp.maximum(degt, 1.0)
                     + maskt * bw_scr[pl.ds(R + r, 1), :])
        cnt = cnt + mask + maskt

    out_ref[...] = acc / jnp.where(cnt == 0.0, 1.0, cnt)


@jax.jit
def kernel(features, adjacencies, adjacencies_t, w, bias, w_t, bias_t):
    grid = (N // BM,)
    return pl.pallas_call(
        _body,
        grid=grid,
        in_specs=[
            pl.BlockSpec((N, D), lambda m: (0, 0)),
            pl.BlockSpec((R, BM, N), lambda m: (0, m, 0)),
            pl.BlockSpec((R, BM, N), lambda m: (0, m, 0)),
            pl.BlockSpec((R, D, D), lambda m: (0, 0, 0)),
            pl.BlockSpec((R, D), lambda m: (0, 0)),
            pl.BlockSpec((R, D, D), lambda m: (0, 0, 0)),
            pl.BlockSpec((R, D), lambda m: (0, 0)),
        ],
        out_specs=pl.BlockSpec((BM, D), lambda m: (m, 0)),
        out_shape=jax.ShapeDtypeStruct((N, D), jnp.float32),
        scratch_shapes=[
            pltpu.VMEM((R, N, D), jnp.bfloat16),
            pltpu.VMEM((R, N, D), jnp.bfloat16),
            pltpu.VMEM((2 * R, D), jnp.float32),
        ],
    )(features, adjacencies, adjacencies_t, w, bias, w_t, bias_t)


# speedup vs baseline: 1.0093x; 1.0093x over previous
import jax
import jax.numpy as jnp
from jax.experimental import pallas as pl
from jax.experimental.pallas import tpu as pltpu

N = 4096
D = 128
R = 2
BM = 256


def _body(feat_ref, adj_ref, adjt_ref, w_ref, b_ref, wt_ref, bt_ref,
          out_ref, g_scr, gt_scr):
    m = pl.program_id(0)

    @pl.when(m == 0)
    def _prologue():
        f = feat_ref[...].astype(jnp.bfloat16)
        for r in range(R):
            g_scr[r] = jnp.dot(f, w_ref[r].astype(jnp.bfloat16),
                               preferred_element_type=jnp.float32).astype(jnp.bfloat16)
            gt_scr[r] = jnp.dot(f, wt_ref[r].astype(jnp.bfloat16),
                                preferred_element_type=jnp.float32).astype(jnp.bfloat16)

    acc = jnp.zeros((BM, D), jnp.float32)
    cnt = jnp.zeros((BM, 1), jnp.float32)
    for r in range(R):
        a = adj_ref[r]
        at = adjt_ref[r]
        y = jnp.dot(a.astype(jnp.bfloat16), g_scr[r],
                    preferred_element_type=jnp.float32)
        yt = jnp.dot(at.astype(jnp.bfloat16), gt_scr[r],
                     preferred_element_type=jnp.float32)
        deg = jnp.sum(a, axis=1, keepdims=True).astype(jnp.float32)
        degt = jnp.sum(at, axis=1, keepdims=True).astype(jnp.float32)
        mask = (deg > 0.0).astype(jnp.float32)
        maskt = (degt > 0.0).astype(jnp.float32)
        bw = jnp.dot(b_ref[pl.ds(r, 1), :], w_ref[r],
                     preferred_element_type=jnp.float32)
        bwt = jnp.dot(bt_ref[pl.ds(r, 1), :], wt_ref[r],
                      preferred_element_type=jnp.float32)
        acc = acc + (y / jnp.maximum(deg, 1.0) + mask * bw
                     + yt / jnp.maximum(degt, 1.0) + maskt * bwt)
        cnt = cnt + mask + maskt

    out_ref[...] = acc / jnp.where(cnt == 0.0, 1.0, cnt)


@jax.jit
def kernel(features, adjacencies, adjacencies_t, w, bias, w_t, bias_t):
    grid = (N // BM,)
    return pl.pallas_call(
        _body,
        grid=grid,
        in_specs=[
            pl.BlockSpec((N, D), lambda m: (0, 0)),
            pl.BlockSpec((R, BM, N), lambda m: (0, m, 0)),
            pl.BlockSpec((R, BM, N), lambda m: (0, m, 0)),
            pl.BlockSpec((R, D, D), lambda m: (0, 0, 0)),
            pl.BlockSpec((R, D), lambda m: (0, 0)),
            pl.BlockSpec((R, D, D), lambda m: (0, 0, 0)),
            pl.BlockSpec((R, D), lambda m: (0, 0)),
        ],
        out_specs=pl.BlockSpec((BM, D), lambda m: (m, 0)),
        out_shape=jax.ShapeDtypeStruct((N, D), jnp.float32),
        scratch_shapes=[
            pltpu.VMEM((R, N, D), jnp.bfloat16),
            pltpu.VMEM((R, N, D), jnp.bfloat16),
        ],
    )(features, adjacencies, adjacencies_t, w, bias, w_t, bias_t)
